# baseline (speedup 1.0000x reference)
import functools

import jax
import jax.numpy as jnp
from jax import lax
from jax.experimental import pallas as pl
from jax.experimental.pallas import tpu as pltpu
from jax.experimental.pallas import tpu_sc as plsc

N = 10000
D = 128
E = 160000
C_FEAT = 256
HW = 112

NC, NS, LANES = 2, 16, 16
NW = NC * NS

NPAD = 10240
VPT = NPAD // NW
VCH = 32

EPW = 5120
EP = NW * EPW
ECH = 32
NECH = EPW // ECH

ACC_ROWS = NPAD
RPT = ACC_ROWS // NS
ZR = 16

BR = 1000
GRID = N // BR

_sc_mesh = plsc.VectorSubcoreMesh(
    core_axis_name="c", subcore_axis_name="s", num_cores=NC, num_subcores=NS)


NVCH = VPT // VCH
VCH4 = 4 * VCH


def _sc_bilinear_body(featT, posx_h, posy_h, g0, g1, g2, g3,
                      posx, posy, ib0, ib1, rows0, rows1,
                      sg0, sg1, sw0, sw1):
    cid = lax.axis_index("c")
    sid = lax.axis_index("s")
    wid = sid * NC + cid
    vbase0 = wid * VPT
    ibs = (ib0, ib1)
    rows = (rows0, rows1)
    sg = (sg0, sg1)
    sw = (sw0, sw1)
    gouts = (g0, g1, g2, g3)

    pltpu.sync_copy(posx_h.at[pl.ds(vbase0, VPT)], posx)
    pltpu.sync_copy(posy_h.at[pl.ds(vbase0, VPT)], posy)

    def compute_idx(c, s):
        for r in range(VCH // LANES):
            lsl = pl.ds(c * VCH + r * LANES, LANES)
            x = posx[lsl] * float(HW - 1)
            y = posy[lsl] * float(HW - 1)
            x0 = x.astype(jnp.int32)
            y0 = y.astype(jnp.int32)
            p = y0 * HW + x0
            ibs[s][pl.ds(r * LANES, LANES)] = p
            ibs[s][pl.ds(VCH + r * LANES, LANES)] = p + 1
            ibs[s][pl.ds(2 * VCH + r * LANES, LANES)] = p + HW
            ibs[s][pl.ds(3 * VCH + r * LANES, LANES)] = p + HW + 1

    def fire_gather(s):
        pltpu.async_copy(featT.at[ibs[s]], rows[s], sg[s])

    def wait_gather(s):
        pltpu.make_async_copy(
            featT.at[pl.ds(0, VCH4)], rows[s], sg[s]).wait()

    def fire_writes(c, s):
        vb = vbase0 + c * VCH
        for k in range(4):
            pltpu.async_copy(
                rows[s].at[pl.ds(k * VCH, VCH)],
                gouts[k].at[pl.ds(vb, VCH)], sw[s])

    def wait_writes(s):
        for _ in range(4):
            pltpu.make_async_copy(
                rows[s].at[pl.ds(0, VCH)], g0.at[pl.ds(0, VCH)],
                sw[s]).wait()

    compute_idx(0, 0)
    fire_gather(0)

    def pair(p, carry):
        c0 = p * 2
        compute_idx(c0 + 1, 1)

        @pl.when(p >= 1)
        def _():
            wait_writes(1)
        fire_gather(1)
        wait_gather(0)
        fire_writes(c0, 0)
        @pl.when(c0 + 2 < NVCH)
        def _():
            compute_idx(c0 + 2, 0)
            wait_writes(0)
            fire_gather(0)
        wait_gather(1)
        fire_writes(c0 + 1, 1)
        return carry

    lax.fori_loop(0, NVCH // 2, pair, 0)
    wait_writes(0)
    wait_writes(1)


_sc_bilinear = functools.partial(
    pl.kernel,
    out_type=[jax.ShapeDtypeStruct((NPAD, C_FEAT), jnp.float32)] * 4,
    mesh=_sc_mesh,
    scratch_types=[
        pltpu.VMEM((VPT,), jnp.float32),
        pltpu.VMEM((VPT,), jnp.float32),
        pltpu.VMEM((VCH4,), jnp.int32),
        pltpu.VMEM((VCH4,), jnp.int32),
        pltpu.VMEM((VCH4, C_FEAT), jnp.float32),
        pltpu.VMEM((VCH4, C_FEAT), jnp.float32),
        pltpu.SemaphoreType.DMA,
        pltpu.SemaphoreType.DMA,
        pltpu.SemaphoreType.DMA,
        pltpu.SemaphoreType.DMA,
    ],
)(_sc_bilinear_body)


NBUF = 8


def _sc_segsum_body(y, srcp, dstp, out, zbuf, idx_s, idx_d,
                    r0, r1, r2, r3, r4, r5, r6, r7, acc,
                    semz, sg0, sg1, sg2, sg3, sg4, sg5, sg6, sg7,
                    ss0, ss1, ss2, ss3, ss4, ss5, ss6, ss7):
    cid = lax.axis_index("c")
    sid = lax.axis_index("s")
    wid = sid * NC + cid
    bufs = (r0, r1, r2, r3, r4, r5, r6, r7)
    sg = (sg0, sg1, sg2, sg3, sg4, sg5, sg6, sg7)
    ss = (ss0, ss1, ss2, ss3, ss4, ss5, ss6, ss7)

    def zloop(i, carry):
        for c in range(D // LANES):
            zbuf[i, pl.ds(c * LANES, LANES)] = jnp.zeros((LANES,), jnp.float32)
        return carry

    lax.fori_loop(0, ZR, zloop, 0)
    rbase = sid * RPT
    for b in range(RPT // ZR):
        pltpu.async_copy(zbuf, acc.at[pl.ds(rbase + b * ZR, ZR)], semz)
    ebase = wid * EPW
    pltpu.sync_copy(srcp.at[pl.ds(ebase, EPW)], idx_s)
    pltpu.sync_copy(dstp.at[pl.ds(ebase, EPW)], idx_d)
    for b in range(RPT // ZR):
        pltpu.make_async_copy(zbuf, acc.at[pl.ds(rbase, ZR)], semz).wait()
    plsc.subcore_barrier()

    for b in range(NBUF):
        pltpu.async_copy(y.at[idx_s.at[pl.ds(b * ECH, ECH)]], bufs[b], sg[b])

    def body(j4, carry):
        for b in range(NBUF):
            j = j4 * NBUF + b
            pltpu.make_async_copy(y.at[pl.ds(0, ECH)], bufs[b], sg[b]).wait()
            pltpu.async_copy(
                bufs[b], acc.at[idx_d.at[pl.ds(j * ECH, ECH)]], ss[b],
                add=True)
            pltpu.make_async_copy(bufs[b], acc.at[pl.ds(0, ECH)], ss[b]).wait()

            @pl.when(j + NBUF < NECH)
            def _():
                pltpu.async_copy(
                    y.at[idx_s.at[pl.ds((j + NBUF) * ECH, ECH)]],
                    bufs[b], sg[b])
        return carry

    lax.fori_loop(0, NECH // NBUF, body, 0)
    plsc.subcore_barrier()
    pltpu.sync_copy(acc.at[pl.ds(rbase, RPT)], out.at[cid, pl.ds(rbase, RPT)])


_sc_segsum = functools.partial(
    pl.kernel,
    out_type=jax.ShapeDtypeStruct((NC, ACC_ROWS, D), jnp.float32),
    mesh=_sc_mesh,
    scratch_types=[
        pltpu.VMEM((ZR, D), jnp.float32),
        pltpu.VMEM((EPW,), jnp.int32),
        pltpu.VMEM((EPW,), jnp.int32),
        pltpu.VMEM((ECH, D), jnp.float32),
        pltpu.VMEM((ECH, D), jnp.float32),
        pltpu.VMEM((ECH, D), jnp.float32),
        pltpu.VMEM((ECH, D), jnp.float32),
        pltpu.VMEM((ECH, D), jnp.float32),
        pltpu.VMEM((ECH, D), jnp.float32),
        pltpu.VMEM((ECH, D), jnp.float32),
        pltpu.VMEM((ECH, D), jnp.float32),
        pltpu.VMEM_SHARED((ACC_ROWS, D), jnp.float32),
    ] + [pltpu.SemaphoreType.DMA] * 17,
)(_sc_segsum_body)


def _tc_l0_body(vf_ref, pos_ref, g0_ref, g1_ref, g2_ref, g3_ref,
                wa_ref, wb_ref, wc_ref, bv_ref, y0_ref, y1_ref):
    pos = pos_ref[...]
    x = pos[:, 0:1] * float(HW - 1)
    y = pos[:, 1:2] * float(HW - 1)
    wx1 = x - jnp.floor(x)
    wy1 = y - jnp.floor(y)
    wx0 = 1.0 - wx1
    wy0 = 1.0 - wy1
    aligned = (wy0 * wx0 * g0_ref[...] + wy0 * wx1 * g1_ref[...]
               + wy1 * wx0 * g2_ref[...] + wy1 * wx1 * g3_ref[...])
    y = (jnp.dot(vf_ref[...], wa_ref[...], preferred_element_type=jnp.float32)
         + jnp.dot(pos_ref[...], wb_ref[...], preferred_element_type=jnp.float32)
         + jnp.dot(aligned, wc_ref[...], preferred_element_type=jnp.float32)
         + bv_ref[...][None, :])
    y0_ref[...] = y[:, :D]
    y1_ref[...] = y[:, D:]


def _tc_l12_body(y0_ref, p0_ref, p1_ref, pos_ref, wa_ref, wb_ref, bv_ref,
                 o0_ref, o1_ref):
    h = jax.nn.relu(y0_ref[...] + p0_ref[0] + p1_ref[0])
    y = (jnp.dot(pos_ref[...], wb_ref[...], preferred_element_type=jnp.float32)
         + jnp.dot(h, wa_ref[...], preferred_element_type=jnp.float32)
         + bv_ref[...][None, :])
    o0_ref[...] = y[:, :D]
    o1_ref[...] = y[:, D:]


def _tc_head_body(y0_ref, p0_ref, p1_ref, pos_ref, wla_ref, wlb_ref, bl_ref,
                  nf_ref, np_ref):
    nf = y0_ref[...] + p0_ref[0] + p1_ref[0]
    off = jnp.tanh(
        jnp.dot(pos_ref[...], wlb_ref[...], preferred_element_type=jnp.float32)
        + jnp.dot(nf, wla_ref[...], preferred_element_type=jnp.float32)
        + bl_ref[...][None, :])
    nf_ref[...] = nf
    np_ref[...] = pos_ref[...] + off


def _row_spec(cols):
    return pl.BlockSpec((BR, cols), lambda i: (i, 0))


def _full_spec(shape):
    nd = len(shape)
    return pl.BlockSpec(shape, lambda i: (0,) * nd)


def _part_spec(k):
    return pl.BlockSpec((1, BR, D), lambda i, _k=k: (_k, i, 0))


def _tc_l0(vfeat, pos, g0, g1, g2, g3, Wa, Wb, Wc, bv):
    return pl.pallas_call(
        _tc_l0_body,
        grid=(GRID,),
        in_specs=[
            _row_spec(D), _row_spec(3),
            _row_spec(C_FEAT), _row_spec(C_FEAT), _row_spec(C_FEAT),
            _row_spec(C_FEAT),
            _full_spec(Wa.shape), _full_spec(Wb.shape), _full_spec(Wc.shape),
            _full_spec(bv.shape),
        ],
        out_specs=[_row_spec(D), _row_spec(D)],
        out_shape=[jax.ShapeDtypeStruct((N, D), jnp.float32)] * 2,
    )(vfeat, pos, g0, g1, g2, g3, Wa, Wb, Wc, bv)


def _tc_l12(y0, parts, pos, Wa, Wb, bv):
    return pl.pallas_call(
        _tc_l12_body,
        grid=(GRID,),
        in_specs=[
            _row_spec(D), _part_spec(0), _part_spec(1), _row_spec(3),
            _full_spec(Wa.shape), _full_spec(Wb.shape), _full_spec(bv.shape),
        ],
        out_specs=[_row_spec(D), _row_spec(D)],
        out_shape=[jax.ShapeDtypeStruct((N, D), jnp.float32)] * 2,
    )(y0, parts, parts, pos, Wa, Wb, bv)


def _tc_head(y0, parts, pos, Wla, Wlb, bl):
    return pl.pallas_call(
        _tc_head_body,
        grid=(GRID,),
        in_specs=[
            _row_spec(D), _part_spec(0), _part_spec(1), _row_spec(3),
            _full_spec(Wla.shape), _full_spec(Wlb.shape), _full_spec(bl.shape),
        ],
        out_specs=[_row_spec(D), _row_spec(3)],
        out_shape=[jax.ShapeDtypeStruct((N, D), jnp.float32),
                   jax.ShapeDtypeStruct((N, 3), jnp.float32)],
    )(y0, parts, parts, pos, Wla, Wlb, bl)


def kernel(back_bone_features, vertex_positions, vertex_features, edge_index,
           W0_0, W1_0, b_0, W0_1, W1_1, b_1, W0_2, W1_2, b_2, W_lin, b_lin):
    f32 = jnp.float32
    featT = back_bone_features[0].reshape(C_FEAT, HW * HW).T
    posx_h = jnp.zeros((NPAD,), f32).at[:N].set(vertex_positions[:, 0])
    posy_h = jnp.zeros((NPAD,), f32).at[:N].set(vertex_positions[:, 1])
    src = edge_index[0]
    dst = edge_index[1]
    pad_i = jnp.arange(EP - E, dtype=jnp.int32)
    srcp = jnp.concatenate([src, pad_i % N])
    dstp = jnp.concatenate([dst, N + pad_i % (ACC_ROWS - N)])

    g0, g1, g2, g3 = _sc_bilinear(featT, posx_h, posy_h)

    Wc0 = jnp.concatenate([W0_0, W1_0], axis=1)
    bv0 = jnp.concatenate([b_0, jnp.zeros((D,), f32)])
    y0_0, y1_0 = _tc_l0(vertex_features, vertex_positions, g0, g1, g2, g3,
                        Wc0[:D], Wc0[D:D + 3], Wc0[D + 3:], bv0)
    p0 = _sc_segsum(y1_0, srcp, dstp)

    Wc1 = jnp.concatenate([W0_1, W1_1], axis=1)
    bv1 = jnp.concatenate([b_1, jnp.zeros((D,), f32)])
    y0_1, y1_1 = _tc_l12(y0_0, p0, vertex_positions, Wc1[3:], Wc1[:3], bv1)
    p1 = _sc_segsum(y1_1, srcp, dstp)

    Wc2 = jnp.concatenate([W0_2, W1_2], axis=1)
    bv2 = jnp.concatenate([b_2, jnp.zeros((D,), f32)])
    y0_2, y1_2 = _tc_l12(y0_1, p1, vertex_positions, Wc2[3:], Wc2[:3], bv2)
    p2 = _sc_segsum(y1_2, srcp, dstp)

    new_features, new_positions = _tc_head(
        y0_2, p2, vertex_positions, W_lin[3:], W_lin[:3], b_lin)
    return (new_positions, new_features)


# speedup vs baseline: 1.0039x; 1.0039x over previous
import functools

import jax
import jax.numpy as jnp
from jax import lax
from jax.experimental import pallas as pl
from jax.experimental.pallas import tpu as pltpu
from jax.experimental.pallas import tpu_sc as plsc

N = 10000
D = 128
E = 160000
C_FEAT = 256
HW = 112

NC, NS, LANES = 2, 16, 16
NW = NC * NS

NPAD = 10240
VPT = NPAD // NW
VCH = 32

EPW = 5120
EP = NW * EPW
ECH = 32
NECH = EPW // ECH

ACC_ROWS = NPAD
RPT = ACC_ROWS // NS
ZR = 16

BR = 1000
GRID = N // BR

_sc_mesh = plsc.VectorSubcoreMesh(
    core_axis_name="c", subcore_axis_name="s", num_cores=NC, num_subcores=NS)


NVCH = VPT // VCH
VCH4 = 4 * VCH


def _sc_bilinear_body(featT, posx_h, posy_h, g0, g1, g2, g3,
                      posx, posy, ib0, ib1, rows0, rows1,
                      sg0, sg1, sw0, sw1):
    cid = lax.axis_index("c")
    sid = lax.axis_index("s")
    wid = sid * NC + cid
    vbase0 = wid * VPT
    ibs = (ib0, ib1)
    rows = (rows0, rows1)
    sg = (sg0, sg1)
    sw = (sw0, sw1)
    gouts = (g0, g1, g2, g3)

    pltpu.sync_copy(posx_h.at[pl.ds(vbase0, VPT)], posx)
    pltpu.sync_copy(posy_h.at[pl.ds(vbase0, VPT)], posy)

    def compute_idx(c, s):
        for r in range(VCH // LANES):
            lsl = pl.ds(c * VCH + r * LANES, LANES)
            x = posx[lsl] * float(HW - 1)
            y = posy[lsl] * float(HW - 1)
            x0 = x.astype(jnp.int32)
            y0 = y.astype(jnp.int32)
            p = y0 * HW + x0
            ibs[s][pl.ds(r * LANES, LANES)] = p
            ibs[s][pl.ds(VCH + r * LANES, LANES)] = p + 1
            ibs[s][pl.ds(2 * VCH + r * LANES, LANES)] = p + HW
            ibs[s][pl.ds(3 * VCH + r * LANES, LANES)] = p + HW + 1

    def fire_gather(s):
        pltpu.async_copy(featT.at[ibs[s]], rows[s], sg[s])

    def wait_gather(s):
        pltpu.make_async_copy(
            featT.at[pl.ds(0, VCH4)], rows[s], sg[s]).wait()

    def fire_writes(c, s):
        vb = vbase0 + c * VCH
        for k in range(4):
            pltpu.async_copy(
                rows[s].at[pl.ds(k * VCH, VCH)],
                gouts[k].at[pl.ds(vb, VCH)], sw[s])

    def wait_writes(s):
        for _ in range(4):
            pltpu.make_async_copy(
                rows[s].at[pl.ds(0, VCH)], g0.at[pl.ds(0, VCH)],
                sw[s]).wait()

    compute_idx(0, 0)
    fire_gather(0)

    def pair(p, carry):
        c0 = p * 2
        compute_idx(c0 + 1, 1)

        @pl.when(p >= 1)
        def _():
            wait_writes(1)
        fire_gather(1)
        wait_gather(0)
        fire_writes(c0, 0)
        @pl.when(c0 + 2 < NVCH)
        def _():
            compute_idx(c0 + 2, 0)
            wait_writes(0)
            fire_gather(0)
        wait_gather(1)
        fire_writes(c0 + 1, 1)
        return carry

    lax.fori_loop(0, NVCH // 2, pair, 0)
    wait_writes(0)
    wait_writes(1)


_sc_bilinear = functools.partial(
    pl.kernel,
    out_type=[jax.ShapeDtypeStruct((NPAD, C_FEAT), jnp.float32)] * 4,
    mesh=_sc_mesh,
    scratch_types=[
        pltpu.VMEM((VPT,), jnp.float32),
        pltpu.VMEM((VPT,), jnp.float32),
        pltpu.VMEM((VCH4,), jnp.int32),
        pltpu.VMEM((VCH4,), jnp.int32),
        pltpu.VMEM((VCH4, C_FEAT), jnp.float32),
        pltpu.VMEM((VCH4, C_FEAT), jnp.float32),
        pltpu.SemaphoreType.DMA,
        pltpu.SemaphoreType.DMA,
        pltpu.SemaphoreType.DMA,
        pltpu.SemaphoreType.DMA,
    ],
)(_sc_bilinear_body)


NBUF = 8


def _sc_segsum_body(y, srcp, dstp, out, zbuf, idx_s, idx_d,
                    r0, r1, r2, r3, r4, r5, r6, r7, acc,
                    semz, sg0, sg1, sg2, sg3, sg4, sg5, sg6, sg7,
                    ss0, ss1, ss2, ss3, ss4, ss5, ss6, ss7):
    cid = lax.axis_index("c")
    sid = lax.axis_index("s")
    wid = sid * NC + cid
    bufs = (r0, r1, r2, r3, r4, r5, r6, r7)
    sg = (sg0, sg1, sg2, sg3, sg4, sg5, sg6, sg7)
    ss = (ss0, ss1, ss2, ss3, ss4, ss5, ss6, ss7)

    def zloop(i, carry):
        for c in range(D // LANES):
            zbuf[i, pl.ds(c * LANES, LANES)] = jnp.zeros((LANES,), jnp.float32)
        return carry

    lax.fori_loop(0, ZR, zloop, 0)
    rbase = sid * RPT
    for b in range(RPT // ZR):
        pltpu.async_copy(zbuf, acc.at[pl.ds(rbase + b * ZR, ZR)], semz)
    ebase = wid * EPW
    pltpu.sync_copy(srcp.at[pl.ds(ebase, EPW)], idx_s)
    pltpu.sync_copy(dstp.at[pl.ds(ebase, EPW)], idx_d)
    for b in range(NBUF):
        pltpu.async_copy(y.at[idx_s.at[pl.ds(b * ECH, ECH)]], bufs[b], sg[b])
    for b in range(RPT // ZR):
        pltpu.make_async_copy(zbuf, acc.at[pl.ds(rbase, ZR)], semz).wait()
    plsc.subcore_barrier()

    def body(j4, carry):
        for b in range(NBUF):
            j = j4 * NBUF + b
            pltpu.make_async_copy(y.at[pl.ds(0, ECH)], bufs[b], sg[b]).wait()
            pltpu.async_copy(
                bufs[b], acc.at[idx_d.at[pl.ds(j * ECH, ECH)]], ss[b],
                add=True)

            @pl.when(j + NBUF < NECH)
            def _():
                pltpu.make_async_copy(
                    bufs[b], acc.at[pl.ds(0, ECH)], ss[b]).wait()
                pltpu.async_copy(
                    y.at[idx_s.at[pl.ds((j + NBUF) * ECH, ECH)]],
                    bufs[b], sg[b])
        return carry

    lax.fori_loop(0, NECH // NBUF, body, 0)
    for b in range(NBUF):
        pltpu.make_async_copy(bufs[b], acc.at[pl.ds(0, ECH)], ss[b]).wait()
    plsc.subcore_barrier()
    pltpu.sync_copy(acc.at[pl.ds(rbase, RPT)], out.at[cid, pl.ds(rbase, RPT)])


_sc_segsum = functools.partial(
    pl.kernel,
    out_type=jax.ShapeDtypeStruct((NC, ACC_ROWS, D), jnp.float32),
    mesh=_sc_mesh,
    scratch_types=[
        pltpu.VMEM((ZR, D), jnp.float32),
        pltpu.VMEM((EPW,), jnp.int32),
        pltpu.VMEM((EPW,), jnp.int32),
        pltpu.VMEM((ECH, D), jnp.float32),
        pltpu.VMEM((ECH, D), jnp.float32),
        pltpu.VMEM((ECH, D), jnp.float32),
        pltpu.VMEM((ECH, D), jnp.float32),
        pltpu.VMEM((ECH, D), jnp.float32),
        pltpu.VMEM((ECH, D), jnp.float32),
        pltpu.VMEM((ECH, D), jnp.float32),
        pltpu.VMEM((ECH, D), jnp.float32),
        pltpu.VMEM_SHARED((ACC_ROWS, D), jnp.float32),
    ] + [pltpu.SemaphoreType.DMA] * 17,
)(_sc_segsum_body)


def _tc_l0_body(vf_ref, pos_ref, g0_ref, g1_ref, g2_ref, g3_ref,
                wa_ref, wb_ref, wc_ref, bv_ref, y0_ref, y1_ref):
    pos = pos_ref[...]
    x = pos[:, 0:1] * float(HW - 1)
    y = pos[:, 1:2] * float(HW - 1)
    wx1 = x - jnp.floor(x)
    wy1 = y - jnp.floor(y)
    wx0 = 1.0 - wx1
    wy0 = 1.0 - wy1
    aligned = (wy0 * wx0 * g0_ref[...] + wy0 * wx1 * g1_ref[...]
               + wy1 * wx0 * g2_ref[...] + wy1 * wx1 * g3_ref[...])
    y = (jnp.dot(vf_ref[...], wa_ref[...], preferred_element_type=jnp.float32)
         + jnp.dot(pos_ref[...], wb_ref[...], preferred_element_type=jnp.float32)
         + jnp.dot(aligned, wc_ref[...], preferred_element_type=jnp.float32)
         + bv_ref[...][None, :])
    y0_ref[...] = y[:, :D]
    y1_ref[...] = y[:, D:]


def _tc_l12_body(y0_ref, p0_ref, p1_ref, pos_ref, wa_ref, wb_ref, bv_ref,
                 o0_ref, o1_ref):
    h = jax.nn.relu(y0_ref[...] + p0_ref[0] + p1_ref[0])
    y = (jnp.dot(pos_ref[...], wb_ref[...], preferred_element_type=jnp.float32)
         + jnp.dot(h, wa_ref[...], preferred_element_type=jnp.float32)
         + bv_ref[...][None, :])
    o0_ref[...] = y[:, :D]
    o1_ref[...] = y[:, D:]


def _tc_head_body(y0_ref, p0_ref, p1_ref, pos_ref, wla_ref, wlb_ref, bl_ref,
                  nf_ref, np_ref):
    nf = y0_ref[...] + p0_ref[0] + p1_ref[0]
    off = jnp.tanh(
        jnp.dot(pos_ref[...], wlb_ref[...], preferred_element_type=jnp.float32)
        + jnp.dot(nf, wla_ref[...], preferred_element_type=jnp.float32)
        + bl_ref[...][None, :])
    nf_ref[...] = nf
    np_ref[...] = pos_ref[...] + off


def _row_spec(cols):
    return pl.BlockSpec((BR, cols), lambda i: (i, 0))


def _full_spec(shape):
    nd = len(shape)
    return pl.BlockSpec(shape, lambda i: (0,) * nd)


def _part_spec(k):
    return pl.BlockSpec((1, BR, D), lambda i, _k=k: (_k, i, 0))


def _tc_l0(vfeat, pos, g0, g1, g2, g3, Wa, Wb, Wc, bv):
    return pl.pallas_call(
        _tc_l0_body,
        grid=(GRID,),
        in_specs=[
            _row_spec(D), _row_spec(3),
            _row_spec(C_FEAT), _row_spec(C_FEAT), _row_spec(C_FEAT),
            _row_spec(C_FEAT),
            _full_spec(Wa.shape), _full_spec(Wb.shape), _full_spec(Wc.shape),
            _full_spec(bv.shape),
        ],
        out_specs=[_row_spec(D), _row_spec(D)],
        out_shape=[jax.ShapeDtypeStruct((N, D), jnp.float32)] * 2,
    )(vfeat, pos, g0, g1, g2, g3, Wa, Wb, Wc, bv)


def _tc_l12(y0, parts, pos, Wa, Wb, bv):
    return pl.pallas_call(
        _tc_l12_body,
        grid=(GRID,),
        in_specs=[
            _row_spec(D), _part_spec(0), _part_spec(1), _row_spec(3),
            _full_spec(Wa.shape), _full_spec(Wb.shape), _full_spec(bv.shape),
        ],
        out_specs=[_row_spec(D), _row_spec(D)],
        out_shape=[jax.ShapeDtypeStruct((N, D), jnp.float32)] * 2,
    )(y0, parts, parts, pos, Wa, Wb, bv)


def _tc_head(y0, parts, pos, Wla, Wlb, bl):
    return pl.pallas_call(
        _tc_head_body,
        grid=(GRID,),
        in_specs=[
            _row_spec(D), _part_spec(0), _part_spec(1), _row_spec(3),
            _full_spec(Wla.shape), _full_spec(Wlb.shape), _full_spec(bl.shape),
        ],
        out_specs=[_row_spec(D), _row_spec(3)],
        out_shape=[jax.ShapeDtypeStruct((N, D), jnp.float32),
                   jax.ShapeDtypeStruct((N, 3), jnp.float32)],
    )(y0, parts, parts, pos, Wla, Wlb, bl)


def kernel(back_bone_features, vertex_positions, vertex_features, edge_index,
           W0_0, W1_0, b_0, W0_1, W1_1, b_1, W0_2, W1_2, b_2, W_lin, b_lin):
    f32 = jnp.float32
    featT = back_bone_features[0].reshape(C_FEAT, HW * HW).T
    posx_h = jnp.zeros((NPAD,), f32).at[:N].set(vertex_positions[:, 0])
    posy_h = jnp.zeros((NPAD,), f32).at[:N].set(vertex_positions[:, 1])
    src = edge_index[0]
    dst = edge_index[1]
    pad_i = jnp.arange(EP - E, dtype=jnp.int32)
    srcp = jnp.concatenate([src, pad_i % N])
    dstp = jnp.concatenate([dst, N + pad_i % (ACC_ROWS - N)])

    g0, g1, g2, g3 = _sc_bilinear(featT, posx_h, posy_h)

    Wc0 = jnp.concatenate([W0_0, W1_0], axis=1)
    bv0 = jnp.concatenate([b_0, jnp.zeros((D,), f32)])
    y0_0, y1_0 = _tc_l0(vertex_features, vertex_positions, g0, g1, g2, g3,
                        Wc0[:D], Wc0[D:D + 3], Wc0[D + 3:], bv0)
    p0 = _sc_segsum(y1_0, srcp, dstp)

    Wc1 = jnp.concatenate([W0_1, W1_1], axis=1)
    bv1 = jnp.concatenate([b_1, jnp.zeros((D,), f32)])
    y0_1, y1_1 = _tc_l12(y0_0, p0, vertex_positions, Wc1[3:], Wc1[:3], bv1)
    p1 = _sc_segsum(y1_1, srcp, dstp)

    Wc2 = jnp.concatenate([W0_2, W1_2], axis=1)
    bv2 = jnp.concatenate([b_2, jnp.zeros((D,), f32)])
    y0_2, y1_2 = _tc_l12(y0_1, p1, vertex_positions, Wc2[3:], Wc2[:3], bv2)
    p2 = _sc_segsum(y1_2, srcp, dstp)

    new_features, new_positions = _tc_head(
        y0_2, p2, vertex_positions, W_lin[3:], W_lin[:3], b_lin)
    return (new_positions, new_features)
